# baseline (speedup 1.0000x reference)
import functools

import jax
import jax.numpy as jnp
from jax import lax
from jax.experimental import pallas as pl
from jax.experimental.pallas import tpu as pltpu
from jax.experimental.pallas import tpu_sc as plsc

N = 10000
F = 128
E = 320000
HID = 128

NPAD = 10240
NPT = NPAD // 16
EB = 128
NBW = 80
ER = 32 * NBW
EPAD = ER * EB
RPS = ER // 16
NBUF = 2
RING = 16


def _deg_body(dst_hbm, w_hbm, deg_out, deg_sh, dstb, wb, dloc, sem):
    c = lax.axis_index("c")
    s = lax.axis_index("s")
    base = s * NPT

    def zero_dloc(i, _):
        dloc[pl.ds(i * 16, 16)] = jnp.zeros((16,), jnp.float32)
        return 0
    lax.fori_loop(0, NPAD // 16, zero_dloc, 0)
    pltpu.sync_copy(dloc.at[pl.ds(0, NPT)], deg_sh.at[pl.ds(base, NPT)])
    plsc.subcore_barrier()

    def deg_chunk(ch, _):
        row0 = pl.multiple_of((c * 16 + s) * (ER // 32) + ch * RING, 16)
        pltpu.sync_copy(dst_hbm.at[pl.ds(row0, RING)], dstb)
        pltpu.sync_copy(w_hbm.at[pl.ds(row0, RING)], wb)

        def fire(i, _):
            pltpu.async_copy(wb.at[i], deg_sh.at[dstb.at[i]], sem, add=True)
            return 0
        lax.fori_loop(0, RING, fire, 0)

        def drain(i, _):
            pltpu.make_async_copy(wb.at[i], deg_sh.at[dstb.at[i]], sem).wait()
            return 0
        lax.fori_loop(0, RING, drain, 0)
        return 0
    lax.fori_loop(0, (ER // 32) // RING, deg_chunk, 0)
    plsc.subcore_barrier()
    pltpu.sync_copy(deg_sh.at[pl.ds(base, NPT)], deg_out.at[c, pl.ds(base, NPT)])


_deg_pass = functools.partial(
    pl.kernel,
    out_type=jax.ShapeDtypeStruct((2, NPAD), jnp.float32),
    mesh=plsc.VectorSubcoreMesh(core_axis_name="c", subcore_axis_name="s",
                                num_cores=2, num_subcores=16),
    compiler_params=pltpu.CompilerParams(needs_layout_passes=False),
    scratch_types=(
        pltpu.VMEM_SHARED((NPAD,), jnp.float32),
        pltpu.VMEM((RING, EB), jnp.int32),
        pltpu.VMEM((RING, EB), jnp.float32),
        pltpu.VMEM((NPAD,), jnp.float32),
        pltpu.SemaphoreType.DMA,
    ),
)(_deg_body)


def _sc_body(y_hbm, src_hbm, dst_hbm, w_hbm, z_out,
             z_sh, rows, srcb, dstb, wb,
             gsems, ssems):
    c = lax.axis_index("c")
    s = lax.axis_index("s")
    base = s * NPT
    erow0 = pl.multiple_of((s * 2 + c) * NBW, 8)

    def zero_rows(i, _):
        rows[0, i >> 3, pl.ds((i & 7) * 16, 16)] = jnp.zeros((16,), jnp.float32)
        return 0
    lax.fori_loop(0, EB * (F // 16), zero_rows, 0)

    for k in range(NPT // EB):
        pltpu.sync_copy(rows.at[0], z_sh.at[pl.ds(base + k * EB, EB)])
    plsc.subcore_barrier()

    pltpu.sync_copy(src_hbm.at[pl.ds(erow0, RING)], srcb)
    pltpu.sync_copy(dst_hbm.at[pl.ds(erow0, RING)], dstb)
    pltpu.sync_copy(w_hbm.at[pl.ds(erow0, RING)], wb)

    def start_gather(j):
        pltpu.async_copy(y_hbm.at[srcb.at[j % RING]], rows.at[j % NBUF],
                         gsems.at[j % NBUF])

    def wait_gather(j):
        pltpu.make_async_copy(y_hbm.at[srcb.at[j % RING]], rows.at[j % NBUF],
                              gsems.at[j % NBUF]).wait()

    def start_scatter(j):
        pltpu.async_copy(rows.at[j % NBUF], z_sh.at[dstb.at[j % RING]],
                         ssems.at[j % NBUF], add=True)

    def wait_scatter(j):
        pltpu.make_async_copy(rows.at[j % NBUF], z_sh.at[dstb.at[j % RING]],
                              ssems.at[j % NBUF]).wait()

    start_gather(0)

    def edge_body(j, _):
        @pl.when(jnp.logical_and((j & 7) == 0, j + 8 < NBW))
        def _():
            row = pl.multiple_of(erow0 + j + 8, 8)
            slot = pl.multiple_of((j + 8) & (RING - 1), 8)
            pltpu.sync_copy(src_hbm.at[pl.ds(row, 8)], srcb.at[pl.ds(slot, 8)])
            pltpu.sync_copy(dst_hbm.at[pl.ds(row, 8)], dstb.at[pl.ds(slot, 8)])
            pltpu.sync_copy(w_hbm.at[pl.ds(row, 8)], wb.at[pl.ds(slot, 8)])

        @pl.when(j >= 1)
        def _():
            wait_scatter(j - 1)

        @pl.when(j + 1 < NBW)
        def _():
            start_gather(j + 1)

        wait_gather(j)
        bsel = j % NBUF
        slot_j = j % RING

        def scale_row(r, _):
            coef = plsc.load_gather(
                wb, [jnp.full((16,), slot_j, jnp.int32),
                     jnp.full((16,), r, jnp.int32)])
            for k in range(F // 16):
                rows[bsel, r, pl.ds(k * 16, 16)] = (
                    rows[bsel, r, pl.ds(k * 16, 16)] * coef)
            return 0
        lax.fori_loop(0, EB, scale_row, 0, unroll=4)

        start_scatter(j)
        return 0
    lax.fori_loop(0, NBW, edge_body, 0)
    wait_scatter(NBW - 1)
    plsc.subcore_barrier()

    pltpu.sync_copy(z_sh.at[pl.ds(base, NPT)], z_out.at[c, pl.ds(base, NPT)])


_sc_pass = functools.partial(
    pl.kernel,
    out_type=jax.ShapeDtypeStruct((2, NPAD, F), jnp.float32),
    mesh=plsc.VectorSubcoreMesh(core_axis_name="c", subcore_axis_name="s",
                                num_cores=2, num_subcores=16),
    compiler_params=pltpu.CompilerParams(needs_layout_passes=False),
    scratch_types=(
        pltpu.VMEM_SHARED((NPAD, F), jnp.float32),
        pltpu.VMEM((NBUF, EB, F), jnp.float32),
        pltpu.VMEM((RING, EB), jnp.int32),
        pltpu.VMEM((RING, EB), jnp.int32),
        pltpu.VMEM((RING, EB), jnp.float32),
        pltpu.SemaphoreType.DMA((NBUF,)),
        pltpu.SemaphoreType.DMA((NBUF,)),
    ),
)(_sc_body)


def _score_body(x_ref, p_ref, out_ref):
    out_ref[...] = jnp.dot(x_ref[...], p_ref[...],
                           preferred_element_type=jnp.float32)


def _scores(x, p_col):
    return pl.pallas_call(
        _score_body,
        out_shape=jax.ShapeDtypeStruct((N, 1), jnp.float32),
    )(x, p_col)


def _gru_body(xt_ref, h0_ref, wih_ref, whh_ref, bih_ref, bhh_ref, out_ref):
    xt = xt_ref[...]
    h0 = h0_ref[...]
    gi = jax.lax.dot_general(xt, wih_ref[...], (((1,), (1,)), ((), ())),
                             preferred_element_type=jnp.float32) + bih_ref[...]
    gh = jax.lax.dot_general(h0, whh_ref[...], (((1,), (1,)), ((), ())),
                             preferred_element_type=jnp.float32) + bhh_ref[...]
    i_r, i_z, i_n = gi[:, :F], gi[:, F:2 * F], gi[:, 2 * F:]
    h_r, h_z, h_n = gh[:, :F], gh[:, F:2 * F], gh[:, 2 * F:]
    r = jax.nn.sigmoid(i_r + h_r)
    z = jax.nn.sigmoid(i_z + h_z)
    n = jnp.tanh(i_n + r * h_n)
    out_ref[...] = (1.0 - z) * n + z * h0


def _gru(x_tilde, h0, W_ih, W_hh, b_ih, b_hh):
    return pl.pallas_call(
        _gru_body,
        out_shape=jax.ShapeDtypeStruct((F, F), jnp.float32),
    )(x_tilde, h0, W_ih, W_hh, b_ih.reshape(1, 3 * F), b_hh.reshape(1, 3 * F))


def _final_body(z0_ref, z1_ref, x_ref, dinv_ref, w_ref, wlin_ref, blin_ref,
                out_ref):
    dinv = dinv_ref[...]
    pre = dinv * (z0_ref[...] + z1_ref[...]) + (dinv * dinv) * x_ref[...]
    h = jnp.dot(pre, w_ref[...], preferred_element_type=jnp.float32)
    h = jnp.maximum(h, 0.0)
    out_ref[...] = jax.lax.dot_general(
        h, wlin_ref[...], (((1,), (1,)), ((), ())),
        preferred_element_type=jnp.float32) + blin_ref[...]


def _final(z0, z1, x, dinv_col, W, W_lin, b_lin):
    R = 2000
    grid = N // R
    return pl.pallas_call(
        _final_body,
        grid=(grid,),
        in_specs=[
            pl.BlockSpec((R, F), lambda i: (i, 0)),
            pl.BlockSpec((R, F), lambda i: (i, 0)),
            pl.BlockSpec((R, F), lambda i: (i, 0)),
            pl.BlockSpec((R, 1), lambda i: (i, 0)),
            pl.BlockSpec((F, F), lambda i: (0, 0)),
            pl.BlockSpec((HID, F), lambda i: (0, 0)),
            pl.BlockSpec((1, HID), lambda i: (0, 0)),
        ],
        out_specs=pl.BlockSpec((R, HID), lambda i: (i, 0)),
        out_shape=jax.ShapeDtypeStruct((N, HID), jnp.float32),
    )(z0, z1, x, dinv_col, W, W_lin, b_lin.reshape(1, HID))


def kernel(x, edge_index, edge_weight, p, W_ih, W_hh, b_ih, b_hh, h0, W_lin, b_lin):
    raw = _scores(x, p.reshape(F, 1))[:, 0] / jnp.linalg.norm(p)
    vals_raw, perm = jax.lax.top_k(raw, F)
    x_tilde = x[perm] * jnp.tanh(vals_raw)[:, None]
    W = _gru(x_tilde, h0, W_ih, W_hh, b_ih, b_hh)
    npad = EPAD - E
    src2 = jnp.concatenate([edge_index[0], jnp.zeros((npad,), jnp.int32)])
    dst2 = jnp.concatenate([edge_index[1], jnp.zeros((npad,), jnp.int32)])
    w2 = jnp.concatenate([edge_weight, jnp.zeros((npad,), jnp.float32)])
    src2d = src2.reshape(ER, EB)
    dst2d = dst2.reshape(ER, EB)
    w2d = w2.reshape(ER, EB)
    deg_parts = _deg_pass(dst2d, w2d)
    dinv = jax.lax.rsqrt(deg_parts[0] + deg_parts[1] + 1.0)[:N]
    y = dinv[:, None] * x
    z_parts = _sc_pass(y, src2d, dst2d, w2d)
    return _final(z_parts[0, :N], z_parts[1, :N], x,
                  dinv.reshape(N, 1), W, W_lin, b_lin)


# speedup vs baseline: 1.0058x; 1.0058x over previous
import functools

import jax
import jax.numpy as jnp
from jax import lax
from jax.experimental import pallas as pl
from jax.experimental.pallas import tpu as pltpu
from jax.experimental.pallas import tpu_sc as plsc

N = 10000
F = 128
E = 320000
HID = 128

NPAD = 10240
NPT = NPAD // 16
EB = 64
NBW = 160
ER = 32 * NBW
EPAD = ER * EB
RPS = ER // 16
NBUF = 4
RING = 16


def _deg_body(dst_hbm, w_hbm, deg_out, deg_sh, dstb, wb, dloc, sem):
    c = lax.axis_index("c")
    s = lax.axis_index("s")
    base = s * NPT

    def zero_dloc(i, _):
        dloc[pl.ds(i * 16, 16)] = jnp.zeros((16,), jnp.float32)
        return 0
    lax.fori_loop(0, NPAD // 16, zero_dloc, 0)
    pltpu.sync_copy(dloc.at[pl.ds(0, NPT)], deg_sh.at[pl.ds(base, NPT)])
    plsc.subcore_barrier()

    def deg_chunk(ch, _):
        row0 = pl.multiple_of((c * 16 + s) * (ER // 32) + ch * RING, 16)
        pltpu.sync_copy(dst_hbm.at[pl.ds(row0, RING)], dstb)
        pltpu.sync_copy(w_hbm.at[pl.ds(row0, RING)], wb)

        def fire(i, _):
            pltpu.async_copy(wb.at[i], deg_sh.at[dstb.at[i]], sem, add=True)
            return 0
        lax.fori_loop(0, RING, fire, 0)

        def drain(i, _):
            pltpu.make_async_copy(wb.at[i], deg_sh.at[dstb.at[i]], sem).wait()
            return 0
        lax.fori_loop(0, RING, drain, 0)
        return 0
    lax.fori_loop(0, (ER // 32) // RING, deg_chunk, 0)
    plsc.subcore_barrier()
    pltpu.sync_copy(deg_sh.at[pl.ds(base, NPT)], deg_out.at[c, pl.ds(base, NPT)])


_deg_pass = functools.partial(
    pl.kernel,
    out_type=jax.ShapeDtypeStruct((2, NPAD), jnp.float32),
    mesh=plsc.VectorSubcoreMesh(core_axis_name="c", subcore_axis_name="s",
                                num_cores=2, num_subcores=16),
    compiler_params=pltpu.CompilerParams(needs_layout_passes=False),
    scratch_types=(
        pltpu.VMEM_SHARED((NPAD,), jnp.float32),
        pltpu.VMEM((RING, EB), jnp.int32),
        pltpu.VMEM((RING, EB), jnp.float32),
        pltpu.VMEM((NPAD,), jnp.float32),
        pltpu.SemaphoreType.DMA,
    ),
)(_deg_body)


def _sc_body(y_hbm, src_hbm, dst_hbm, w_hbm, z_out,
             z_sh, rows, srcb, dstb, wb,
             gsems, ssems):
    c = lax.axis_index("c")
    s = lax.axis_index("s")
    base = s * NPT
    erow0 = pl.multiple_of((s * 2 + c) * NBW, 8)

    def zero_rows(i, _):
        rows[0, i >> 3, pl.ds((i & 7) * 16, 16)] = jnp.zeros((16,), jnp.float32)
        return 0
    lax.fori_loop(0, EB * (F // 16), zero_rows, 0)

    for k in range(NPT // EB):
        pltpu.sync_copy(rows.at[0], z_sh.at[pl.ds(base + k * EB, EB)])
    plsc.subcore_barrier()

    pltpu.sync_copy(src_hbm.at[pl.ds(erow0, RING)], srcb)
    pltpu.sync_copy(dst_hbm.at[pl.ds(erow0, RING)], dstb)
    pltpu.sync_copy(w_hbm.at[pl.ds(erow0, RING)], wb)

    def start_gather(j):
        pltpu.async_copy(y_hbm.at[srcb.at[j % RING]], rows.at[j % NBUF],
                         gsems.at[j % NBUF])

    def wait_gather(j):
        pltpu.make_async_copy(y_hbm.at[srcb.at[j % RING]], rows.at[j % NBUF],
                              gsems.at[j % NBUF]).wait()

    def start_scatter(j):
        pltpu.async_copy(rows.at[j % NBUF], z_sh.at[dstb.at[j % RING]],
                         ssems.at[j % NBUF], add=True)

    def wait_scatter(j):
        pltpu.make_async_copy(rows.at[j % NBUF], z_sh.at[dstb.at[j % RING]],
                              ssems.at[j % NBUF]).wait()

    start_gather(0)
    start_gather(1)

    def edge_body(j, _):
        @pl.when(jnp.logical_and((j & 7) == 0, j + 8 < NBW))
        def _():
            row = pl.multiple_of(erow0 + j + 8, 8)
            slot = pl.multiple_of((j + 8) & (RING - 1), 8)
            pltpu.sync_copy(src_hbm.at[pl.ds(row, 8)], srcb.at[pl.ds(slot, 8)])
            pltpu.sync_copy(dst_hbm.at[pl.ds(row, 8)], dstb.at[pl.ds(slot, 8)])
            pltpu.sync_copy(w_hbm.at[pl.ds(row, 8)], wb.at[pl.ds(slot, 8)])

        @pl.when(j >= NBUF - 2)
        def _():
            wait_scatter(j - (NBUF - 2))

        @pl.when(j + 2 < NBW)
        def _():
            start_gather(j + 2)

        wait_gather(j)
        bsel = j % NBUF
        slot_j = j % RING

        def scale_row(r, _):
            coef = plsc.load_gather(
                wb, [jnp.full((16,), slot_j, jnp.int32),
                     jnp.full((16,), r, jnp.int32)])
            for k in range(F // 16):
                rows[bsel, r, pl.ds(k * 16, 16)] = (
                    rows[bsel, r, pl.ds(k * 16, 16)] * coef)
            return 0
        lax.fori_loop(0, EB, scale_row, 0, unroll=4)

        start_scatter(j)
        return 0
    lax.fori_loop(0, NBW, edge_body, 0)
    for t in range(NBUF - 2, 0, -1):
        wait_scatter(NBW - t)
    plsc.subcore_barrier()

    pltpu.sync_copy(z_sh.at[pl.ds(base, NPT)], z_out.at[c, pl.ds(base, NPT)])


_sc_pass = functools.partial(
    pl.kernel,
    out_type=jax.ShapeDtypeStruct((2, NPAD, F), jnp.float32),
    mesh=plsc.VectorSubcoreMesh(core_axis_name="c", subcore_axis_name="s",
                                num_cores=2, num_subcores=16),
    compiler_params=pltpu.CompilerParams(needs_layout_passes=False),
    scratch_types=(
        pltpu.VMEM_SHARED((NPAD, F), jnp.float32),
        pltpu.VMEM((NBUF, EB, F), jnp.float32),
        pltpu.VMEM((RING, EB), jnp.int32),
        pltpu.VMEM((RING, EB), jnp.int32),
        pltpu.VMEM((RING, EB), jnp.float32),
        pltpu.SemaphoreType.DMA((NBUF,)),
        pltpu.SemaphoreType.DMA((NBUF,)),
    ),
)(_sc_body)


def _score_body(x_ref, p_ref, out_ref):
    out_ref[...] = jnp.dot(x_ref[...], p_ref[...],
                           preferred_element_type=jnp.float32)


def _scores(x, p_col):
    return pl.pallas_call(
        _score_body,
        out_shape=jax.ShapeDtypeStruct((N, 1), jnp.float32),
    )(x, p_col)


def _gru_body(xt_ref, h0_ref, wih_ref, whh_ref, bih_ref, bhh_ref, out_ref):
    xt = xt_ref[...]
    h0 = h0_ref[...]
    gi = jax.lax.dot_general(xt, wih_ref[...], (((1,), (1,)), ((), ())),
                             preferred_element_type=jnp.float32) + bih_ref[...]
    gh = jax.lax.dot_general(h0, whh_ref[...], (((1,), (1,)), ((), ())),
                             preferred_element_type=jnp.float32) + bhh_ref[...]
    i_r, i_z, i_n = gi[:, :F], gi[:, F:2 * F], gi[:, 2 * F:]
    h_r, h_z, h_n = gh[:, :F], gh[:, F:2 * F], gh[:, 2 * F:]
    r = jax.nn.sigmoid(i_r + h_r)
    z = jax.nn.sigmoid(i_z + h_z)
    n = jnp.tanh(i_n + r * h_n)
    out_ref[...] = (1.0 - z) * n + z * h0


def _gru(x_tilde, h0, W_ih, W_hh, b_ih, b_hh):
    return pl.pallas_call(
        _gru_body,
        out_shape=jax.ShapeDtypeStruct((F, F), jnp.float32),
    )(x_tilde, h0, W_ih, W_hh, b_ih.reshape(1, 3 * F), b_hh.reshape(1, 3 * F))


def _final_body(z0_ref, z1_ref, x_ref, dinv_ref, w_ref, wlin_ref, blin_ref,
                out_ref):
    dinv = dinv_ref[...]
    pre = dinv * (z0_ref[...] + z1_ref[...]) + (dinv * dinv) * x_ref[...]
    h = jnp.dot(pre, w_ref[...], preferred_element_type=jnp.float32)
    h = jnp.maximum(h, 0.0)
    out_ref[...] = jax.lax.dot_general(
        h, wlin_ref[...], (((1,), (1,)), ((), ())),
        preferred_element_type=jnp.float32) + blin_ref[...]


def _final(z0, z1, x, dinv_col, W, W_lin, b_lin):
    R = 2000
    grid = N // R
    return pl.pallas_call(
        _final_body,
        grid=(grid,),
        in_specs=[
            pl.BlockSpec((R, F), lambda i: (i, 0)),
            pl.BlockSpec((R, F), lambda i: (i, 0)),
            pl.BlockSpec((R, F), lambda i: (i, 0)),
            pl.BlockSpec((R, 1), lambda i: (i, 0)),
            pl.BlockSpec((F, F), lambda i: (0, 0)),
            pl.BlockSpec((HID, F), lambda i: (0, 0)),
            pl.BlockSpec((1, HID), lambda i: (0, 0)),
        ],
        out_specs=pl.BlockSpec((R, HID), lambda i: (i, 0)),
        out_shape=jax.ShapeDtypeStruct((N, HID), jnp.float32),
    )(z0, z1, x, dinv_col, W, W_lin, b_lin.reshape(1, HID))


def kernel(x, edge_index, edge_weight, p, W_ih, W_hh, b_ih, b_hh, h0, W_lin, b_lin):
    raw = _scores(x, p.reshape(F, 1))[:, 0] / jnp.linalg.norm(p)
    vals_raw, perm = jax.lax.top_k(raw, F)
    x_tilde = x[perm] * jnp.tanh(vals_raw)[:, None]
    W = _gru(x_tilde, h0, W_ih, W_hh, b_ih, b_hh)
    npad = EPAD - E
    src2 = jnp.concatenate([edge_index[0], jnp.zeros((npad,), jnp.int32)])
    dst2 = jnp.concatenate([edge_index[1], jnp.zeros((npad,), jnp.int32)])
    w2 = jnp.concatenate([edge_weight, jnp.zeros((npad,), jnp.float32)])
    src2d = src2.reshape(ER, EB)
    dst2d = dst2.reshape(ER, EB)
    w2d = w2.reshape(ER, EB)
    deg_parts = _deg_pass(dst2d, w2d)
    dinv = jax.lax.rsqrt(deg_parts[0] + deg_parts[1] + 1.0)[:N]
    y = dinv[:, None] * x
    z_parts = _sc_pass(y, src2d, dst2d, w2d)
    return _final(z_parts[0, :N], z_parts[1, :N], x,
                  dinv.reshape(N, 1), W, W_lin, b_lin)


# speedup vs baseline: 1.0435x; 1.0375x over previous
import functools

import jax
import jax.numpy as jnp
from jax import lax
from jax.experimental import pallas as pl
from jax.experimental.pallas import tpu as pltpu
from jax.experimental.pallas import tpu_sc as plsc

N = 10000
F = 128
E = 320000
HID = 128

NPAD = 10240
NPT = NPAD // 16
EB = 64
NBW = 160
ER = 32 * NBW
EPAD = ER * EB
RPS = ER // 16
NBUF = 4
RING = 16


def _deg_body(dst_hbm, w_hbm, deg_out, deg_sh, dstb, wb, dloc, sem):
    c = lax.axis_index("c")
    s = lax.axis_index("s")
    base = s * NPT

    def zero_dloc(i, _):
        dloc[pl.ds(i * 16, 16)] = jnp.zeros((16,), jnp.float32)
        return 0
    lax.fori_loop(0, NPAD // 16, zero_dloc, 0)
    pltpu.sync_copy(dloc.at[pl.ds(0, NPT)], deg_sh.at[pl.ds(base, NPT)])
    plsc.subcore_barrier()

    def deg_chunk(ch, _):
        row0 = pl.multiple_of((c * 16 + s) * (ER // 32) + ch * RING, 16)
        pltpu.sync_copy(dst_hbm.at[pl.ds(row0, RING)], dstb)
        pltpu.sync_copy(w_hbm.at[pl.ds(row0, RING)], wb)

        def fire(i, _):
            pltpu.async_copy(wb.at[i], deg_sh.at[dstb.at[i]], sem, add=True)
            return 0
        lax.fori_loop(0, RING, fire, 0)

        def drain(i, _):
            pltpu.make_async_copy(wb.at[i], deg_sh.at[dstb.at[i]], sem).wait()
            return 0
        lax.fori_loop(0, RING, drain, 0)
        return 0
    lax.fori_loop(0, (ER // 32) // RING, deg_chunk, 0)
    plsc.subcore_barrier()
    pltpu.sync_copy(deg_sh.at[pl.ds(base, NPT)], deg_out.at[c, pl.ds(base, NPT)])


_deg_pass = functools.partial(
    pl.kernel,
    out_type=jax.ShapeDtypeStruct((2, NPAD), jnp.float32),
    mesh=plsc.VectorSubcoreMesh(core_axis_name="c", subcore_axis_name="s",
                                num_cores=2, num_subcores=16),
    compiler_params=pltpu.CompilerParams(needs_layout_passes=False),
    scratch_types=(
        pltpu.VMEM_SHARED((NPAD,), jnp.float32),
        pltpu.VMEM((RING, EB), jnp.int32),
        pltpu.VMEM((RING, EB), jnp.float32),
        pltpu.VMEM((NPAD,), jnp.float32),
        pltpu.SemaphoreType.DMA,
    ),
)(_deg_body)


def _sc_body(y_hbm, src_hbm, dst_hbm, w_hbm, z_out,
             z_sh, rows, srcb, dstb, wb,
             gsems, ssems):
    c = lax.axis_index("c")
    s = lax.axis_index("s")
    base = s * NPT
    erow0 = pl.multiple_of((s * 2 + c) * NBW, 8)

    def zero_rows(i, _):
        rows[0, i >> 3, pl.ds((i & 7) * 16, 16)] = jnp.zeros((16,), jnp.float32)
        return 0
    lax.fori_loop(0, EB * (F // 16), zero_rows, 0)

    for k in range(NPT // EB):
        pltpu.sync_copy(rows.at[0], z_sh.at[pl.ds(base + k * EB, EB)])
    plsc.subcore_barrier()

    pltpu.sync_copy(src_hbm.at[pl.ds(erow0, RING)], srcb)
    pltpu.sync_copy(dst_hbm.at[pl.ds(erow0, RING)], dstb)
    pltpu.sync_copy(w_hbm.at[pl.ds(erow0, RING)], wb)

    def start_gather(j):
        pltpu.async_copy(y_hbm.at[srcb.at[j % RING]], rows.at[j % NBUF],
                         gsems.at[j % NBUF])

    def wait_gather(j):
        pltpu.make_async_copy(y_hbm.at[srcb.at[j % RING]], rows.at[j % NBUF],
                              gsems.at[j % NBUF]).wait()

    def start_scatter(j):
        pltpu.async_copy(rows.at[j % NBUF], z_sh.at[dstb.at[j % RING]],
                         ssems.at[j % NBUF], add=True)

    def wait_scatter(j):
        pltpu.make_async_copy(rows.at[j % NBUF], z_sh.at[dstb.at[j % RING]],
                              ssems.at[j % NBUF]).wait()

    start_gather(0)
    start_gather(1)

    def edge_body(j, _):
        @pl.when(jnp.logical_and((j & 7) == 0, j + 8 < NBW))
        def _():
            row = pl.multiple_of(erow0 + j + 8, 8)
            slot = pl.multiple_of((j + 8) & (RING - 1), 8)
            pltpu.sync_copy(src_hbm.at[pl.ds(row, 8)], srcb.at[pl.ds(slot, 8)])
            pltpu.sync_copy(dst_hbm.at[pl.ds(row, 8)], dstb.at[pl.ds(slot, 8)])
            pltpu.sync_copy(w_hbm.at[pl.ds(row, 8)], wb.at[pl.ds(slot, 8)])

        @pl.when(j >= NBUF - 2)
        def _():
            wait_scatter(j - (NBUF - 2))

        @pl.when(j + 2 < NBW)
        def _():
            start_gather(j + 2)

        wait_gather(j)
        bsel = j % NBUF
        slot_j = j % RING

        def scale_row(r, _):
            coef = plsc.load_gather(
                wb, [jnp.full((16,), slot_j, jnp.int32),
                     jnp.full((16,), r, jnp.int32)])
            for k in range(F // 16):
                rows[bsel, r, pl.ds(k * 16, 16)] = (
                    rows[bsel, r, pl.ds(k * 16, 16)] * coef)
            return 0
        lax.fori_loop(0, EB, scale_row, 0, unroll=4)

        start_scatter(j)
        return 0
    lax.fori_loop(0, NBW, edge_body, 0)
    for t in range(NBUF - 2, 0, -1):
        wait_scatter(NBW - t)
    plsc.subcore_barrier()

    pltpu.sync_copy(z_sh.at[pl.ds(base, NPT)], z_out.at[c, pl.ds(base, NPT)])


_sc_pass = functools.partial(
    pl.kernel,
    out_type=jax.ShapeDtypeStruct((2, NPAD, F), jnp.float32),
    mesh=plsc.VectorSubcoreMesh(core_axis_name="c", subcore_axis_name="s",
                                num_cores=2, num_subcores=16),
    compiler_params=pltpu.CompilerParams(needs_layout_passes=False),
    scratch_types=(
        pltpu.VMEM_SHARED((NPAD, F), jnp.float32),
        pltpu.VMEM((NBUF, EB, F), jnp.float32),
        pltpu.VMEM((RING, EB), jnp.int32),
        pltpu.VMEM((RING, EB), jnp.int32),
        pltpu.VMEM((RING, EB), jnp.float32),
        pltpu.SemaphoreType.DMA((NBUF,)),
        pltpu.SemaphoreType.DMA((NBUF,)),
    ),
)(_sc_body)


def _gru_body(xt_ref, h0_ref, wih_ref, whh_ref, bih_ref, bhh_ref, out_ref):
    xt = xt_ref[...]
    h0 = h0_ref[...]
    gi = jax.lax.dot_general(xt, wih_ref[...], (((1,), (1,)), ((), ())),
                             preferred_element_type=jnp.float32) + bih_ref[...]
    gh = jax.lax.dot_general(h0, whh_ref[...], (((1,), (1,)), ((), ())),
                             preferred_element_type=jnp.float32) + bhh_ref[...]
    i_r, i_z, i_n = gi[:, :F], gi[:, F:2 * F], gi[:, 2 * F:]
    h_r, h_z, h_n = gh[:, :F], gh[:, F:2 * F], gh[:, 2 * F:]
    r = jax.nn.sigmoid(i_r + h_r)
    z = jax.nn.sigmoid(i_z + h_z)
    n = jnp.tanh(i_n + r * h_n)
    out_ref[...] = (1.0 - z) * n + z * h0


def _gru(x_tilde, h0, W_ih, W_hh, b_ih, b_hh):
    return pl.pallas_call(
        _gru_body,
        out_shape=jax.ShapeDtypeStruct((F, F), jnp.float32),
    )(x_tilde, h0, W_ih, W_hh, b_ih.reshape(1, 3 * F), b_hh.reshape(1, 3 * F))


def _final_body(z0_ref, z1_ref, x_ref, dinv_ref, w_ref, wlin_ref, blin_ref,
                out_ref):
    dinv = dinv_ref[...]
    pre = dinv * (z0_ref[...] + z1_ref[...]) + (dinv * dinv) * x_ref[...]
    h = jnp.dot(pre, w_ref[...], preferred_element_type=jnp.float32)
    h = jnp.maximum(h, 0.0)
    out_ref[...] = jax.lax.dot_general(
        h, wlin_ref[...], (((1,), (1,)), ((), ())),
        preferred_element_type=jnp.float32) + blin_ref[...]


def _final(z0, z1, x, dinv_col, W, W_lin, b_lin):
    R = 2000
    grid = N // R
    return pl.pallas_call(
        _final_body,
        grid=(grid,),
        in_specs=[
            pl.BlockSpec((R, F), lambda i: (i, 0)),
            pl.BlockSpec((R, F), lambda i: (i, 0)),
            pl.BlockSpec((R, F), lambda i: (i, 0)),
            pl.BlockSpec((R, 1), lambda i: (i, 0)),
            pl.BlockSpec((F, F), lambda i: (0, 0)),
            pl.BlockSpec((HID, F), lambda i: (0, 0)),
            pl.BlockSpec((1, HID), lambda i: (0, 0)),
        ],
        out_specs=pl.BlockSpec((R, HID), lambda i: (i, 0)),
        out_shape=jax.ShapeDtypeStruct((N, HID), jnp.float32),
    )(z0, z1, x, dinv_col, W, W_lin, b_lin.reshape(1, HID))


def kernel(x, edge_index, edge_weight, p, W_ih, W_hh, b_ih, b_hh, h0, W_lin, b_lin):
    score = jnp.tanh((x @ p) / jnp.linalg.norm(p))
    vals, perm = jax.lax.top_k(score, F)
    x_tilde = x[perm] * vals[:, None]
    W = _gru(x_tilde, h0, W_ih, W_hh, b_ih, b_hh)
    npad = EPAD - E
    src2 = jnp.concatenate([edge_index[0], jnp.zeros((npad,), jnp.int32)])
    dst2 = jnp.concatenate([edge_index[1], jnp.zeros((npad,), jnp.int32)])
    w2 = jnp.concatenate([edge_weight, jnp.zeros((npad,), jnp.float32)])
    src2d = src2.reshape(ER, EB)
    dst2d = dst2.reshape(ER, EB)
    w2d = w2.reshape(ER, EB)
    deg_parts = _deg_pass(dst2d, w2d)
    dinv = jax.lax.rsqrt(deg_parts[0] + deg_parts[1] + 1.0)[:N]
    y = dinv[:, None] * x
    z_parts = _sc_pass(y, src2d, dst2d, w2d)
    return _final(z_parts[0, :N], z_parts[1, :N], x,
                  dinv.reshape(N, 1), W, W_lin, b_lin)
